# baseline (speedup 1.0000x reference)
import jax
import jax.numpy as jnp
from jax import lax
from jax.experimental import pallas as pl
from jax.experimental.pallas import tpu as pltpu

N, DEG, D, COORD = 10000, 32, 128, 3
BN = 400
HB = BN // 2


def _mlp(ef, hh, W1, b1, W2, b2):
    h1 = (jnp.dot(hh, W1[:D, :], preferred_element_type=jnp.float32)
          + jnp.dot(ef, W1[D:, :], preferred_element_type=jnp.float32)
          + b1)
    h1 = h1 * jax.nn.sigmoid(h1)
    return hh + jnp.dot(h1, W2, preferred_element_type=jnp.float32) + b2


def _body(xT_ref, hh_ref, tT_ref, ea_ref, eb_ref, W1_ref, b1_ref, W2_ref,
          b2_ref, coordT_ref, h_ref):
    @pl.when(pl.program_id(0) == 0)
    def _():
        t = jnp.clip(tT_ref[...], -1000.0, 1000.0)
        m = jnp.sum(t, axis=1) * (1.0 / DEG)
        coordT_ref[...] = jnp.clip(xT_ref[...], -1000.0, 1000.0) + m

    efa = jnp.sum(ea_ref[...].reshape(HB, DEG, D), axis=1)
    efb = jnp.sum(eb_ref[...].reshape(HB, DEG, D), axis=1)
    W1, b1 = W1_ref[...], b1_ref[...]
    W2, b2 = W2_ref[...], b2_ref[...]
    h_ref[0:HB, :] = _mlp(efa, hh_ref[0:HB, :], W1, b1, W2, b2)
    h_ref[HB:BN, :] = _mlp(efb, hh_ref[HB:BN, :], W1, b1, W2, b2)


def kernel(x, hh, trans, edge_feature, W1, b1, W2, b2):
    xT = x.T
    tT = trans.transpose(2, 1, 0)
    e2 = edge_feature.reshape(N * DEG, D)
    coordT, h = pl.pallas_call(
        _body,
        grid=(N // BN,),
        in_specs=[
            pl.BlockSpec((COORD, N), lambda i: (0, 0)),
            pl.BlockSpec((BN, D), lambda i: (i, 0)),
            pl.BlockSpec((COORD, DEG, N), lambda i: (0, 0, 0)),
            pl.BlockSpec((HB * DEG, D), lambda i: (2 * i, 0)),
            pl.BlockSpec((HB * DEG, D), lambda i: (2 * i + 1, 0)),
            pl.BlockSpec((2 * D, D), lambda i: (0, 0)),
            pl.BlockSpec((1, D), lambda i: (0, 0)),
            pl.BlockSpec((D, D), lambda i: (0, 0)),
            pl.BlockSpec((1, D), lambda i: (0, 0)),
        ],
        out_specs=[
            pl.BlockSpec((COORD, N), lambda i: (0, 0)),
            pl.BlockSpec((BN, D), lambda i: (i, 0)),
        ],
        out_shape=[
            jax.ShapeDtypeStruct((COORD, N), jnp.float32),
            jax.ShapeDtypeStruct((N, D), jnp.float32),
        ],
        compiler_params=pltpu.CompilerParams(
            dimension_semantics=("parallel",),
        ),
    )(xT, hh, tT, e2, e2, W1, b1.reshape(1, D), W2, b2.reshape(1, D))
    return coordT.T, h


# speedup vs baseline: 1.0266x; 1.0266x over previous
import jax
import jax.numpy as jnp
from jax import lax
from jax.experimental import pallas as pl
from jax.experimental.pallas import tpu as pltpu

N, DEG, D, COORD = 10000, 32, 128, 3
BN = 400


def _body(xT_ref, hh_ref, tT_ref, e_ref, W1_ref, b1_ref, W2_ref, b2_ref,
          coordT_ref, h_ref):
    @pl.when(pl.program_id(0) == 0)
    def _():
        t = jnp.clip(tT_ref[...], -1000.0, 1000.0)
        m = jnp.sum(t, axis=1) * (1.0 / DEG)
        coordT_ref[...] = jnp.clip(xT_ref[...], -1000.0, 1000.0) + m

    ef = jnp.sum(e_ref[...].reshape(BN, DEG, D), axis=1)
    hh = hh_ref[...]
    W1 = W1_ref[...]
    h1 = (jnp.dot(hh, W1[:D, :], preferred_element_type=jnp.float32)
          + jnp.dot(ef, W1[D:, :], preferred_element_type=jnp.float32)
          + b1_ref[...])
    h1 = h1 * jax.nn.sigmoid(h1)
    h_ref[...] = (hh
                  + jnp.dot(h1, W2_ref[...], preferred_element_type=jnp.float32)
                  + b2_ref[...])


def kernel(x, hh, trans, edge_feature, W1, b1, W2, b2):
    xT = x.T
    tT = trans.transpose(2, 1, 0)
    e2 = edge_feature.reshape(N * DEG, D)
    coordT, h = pl.pallas_call(
        _body,
        grid=(N // BN,),
        in_specs=[
            pl.BlockSpec((COORD, N), lambda i: (0, 0)),
            pl.BlockSpec((BN, D), lambda i: (i, 0)),
            pl.BlockSpec((COORD, DEG, N), lambda i: (0, 0, 0)),
            pl.BlockSpec((BN * DEG, D), lambda i: (i, 0)),
            pl.BlockSpec((2 * D, D), lambda i: (0, 0)),
            pl.BlockSpec((1, D), lambda i: (0, 0)),
            pl.BlockSpec((D, D), lambda i: (0, 0)),
            pl.BlockSpec((1, D), lambda i: (0, 0)),
        ],
        out_specs=[
            pl.BlockSpec((COORD, N), lambda i: (0, 0)),
            pl.BlockSpec((BN, D), lambda i: (i, 0)),
        ],
        out_shape=[
            jax.ShapeDtypeStruct((COORD, N), jnp.float32),
            jax.ShapeDtypeStruct((N, D), jnp.float32),
        ],
        compiler_params=pltpu.CompilerParams(
            dimension_semantics=("parallel",),
        ),
    )(xT, hh, tT, e2, W1, b1.reshape(1, D), W2, b2.reshape(1, D))
    return coordT.T, h
